# Initial kernel scaffold; baseline (speedup 1.0000x reference)
import jax
import jax.numpy as jnp
from jax.experimental import pallas as pl


def kernel(x, y):
    raise NotImplementedError("write your pallas kernel here")


# speedup vs baseline: 1.0941x; 1.0941x over previous
import jax
import jax.numpy as jnp
from jax.experimental import pallas as pl

_ROWS, _COLS = 16384, 1024
_BLK = 512


def _copy_patch(x_ref, y_ref, o_ref):
    i = pl.program_id(0)

    @pl.when(i > 0)
    def _plain():
        o_ref[...] = x_ref[...]

    @pl.when(i == 0)
    def _patched():
        blk = x_ref[...]
        r = jax.lax.broadcasted_iota(jnp.int32, blk.shape, 0)
        c = jax.lax.broadcasted_iota(jnp.int32, blk.shape, 1)
        o_ref[...] = jnp.where((r == 0) & (c == 3), y_ref[0, 2], blk)


def kernel(x, y):
    return pl.pallas_call(
        _copy_patch,
        grid=(_ROWS // _BLK,),
        in_specs=[
            pl.BlockSpec((_BLK, _COLS), lambda i: (i, 0)),
            pl.BlockSpec((8, _COLS), lambda i: (0, 0)),
        ],
        out_specs=pl.BlockSpec((_BLK, _COLS), lambda i: (i, 0)),
        out_shape=jax.ShapeDtypeStruct((_ROWS, _COLS), x.dtype),
    )(x, y)
